# baseline (speedup 1.0000x reference)
import functools

import jax
import jax.numpy as jnp
from jax import lax
from jax.experimental import pallas as pl
from jax.experimental.pallas import tpu as pltpu
from jax.experimental.pallas import tpu_sc as plsc

_NC = 19
_H = 512
_W = 512
_B = 8
_BH = 256
_GH = _H // _BH

_NPIX = _B * _H * _W
_NW = 32
_CHUNK = _NPIX // _NW
_LANES = 16
_HBINS = _NC * _NC * _LANES
_UNROLL = 8


def _argmax_kernel(x_ref, y_ref, out_ref):
    xb = x_ref[0]
    yb = y_ref[0]

    m = xb[0]
    arg = jnp.zeros((_BH, _W), jnp.int32)
    for c in range(1, _NC):
        v = xb[c]
        gt = v > m
        m = jnp.where(gt, v, m)
        arg = jnp.where(gt, c, arg)

    out_ref[0] = (arg * _NC + yb) * _LANES


_NSUB = 4
_BHALF = _B // 2
_RPB = _BHALF * _H // _NW
_CGRP = _W // _LANES


def _hist_sc_kernel(pk_hbm, out_hbm, idx_v, hist0, histx, sem):
    c = lax.axis_index("c")
    s = lax.axis_index("s")
    wid = s * 2 + c
    rows_per_b = _H // _RPB
    b = wid // rows_per_b
    r0 = (wid % rows_per_b) * _RPB
    cp = pltpu.async_copy(pk_hbm.at[b, pl.ds(r0, _RPB)], idx_v, sem)

    zero = jnp.zeros((_LANES,), jnp.int32)

    @plsc.parallel_loop(0, _HBINS // _LANES, unroll=4)
    def _zero(i):
        hist0[pl.ds(i * _LANES, _LANES)] = zero
        for k in range(_NSUB - 1):
            histx[pl.ds(k * _HBINS + i * _LANES, _LANES)] = zero

    cp.wait()

    ones = jnp.ones((_LANES,), jnp.int32)
    iota = lax.iota(jnp.int32, _LANES)
    offs = [iota + (k * _HBINS) for k in range(_NSUB - 1)]

    @plsc.parallel_loop(0, _RPB, unroll=2)
    def _accum(r):
        for j in range(_CGRP):
            v = idx_v[r, pl.ds(j * _LANES, _LANES)]
            k = j % _NSUB
            if k == 0:
                plsc.addupdate_scatter(hist0, [v + iota], ones)
            else:
                plsc.addupdate_scatter(histx, [v + offs[k - 1]], ones)

    @plsc.parallel_loop(0, _HBINS // _LANES, unroll=4)
    def _fold(i):
        o = i * _LANES
        a = histx[pl.ds(o, _LANES)] + histx[pl.ds(_HBINS + o, _LANES)]
        b2 = hist0[pl.ds(o, _LANES)] + histx[pl.ds(2 * _HBINS + o, _LANES)]
        hist0[pl.ds(o, _LANES)] = a + b2

    pltpu.sync_copy(hist0, out_hbm.at[wid])


def _epilogue_kernel(ha_ref, hb_ref, out_ref):
    hf = ha_ref[...].astype(jnp.float32) + hb_ref[...].astype(jnp.float32)
    conf = jnp.sum(jnp.sum(hf, axis=3), axis=0)

    ii = lax.broadcasted_iota(jnp.int32, (_NC, _NC), 0)
    jj = lax.broadcasted_iota(jnp.int32, (_NC, _NC), 1)
    eyem = ii == jj

    rowm = jnp.sum(conf, axis=1, keepdims=True)
    colm = jnp.sum(conf, axis=0, keepdims=True)
    union = rowm + colm - conf + jnp.float32(1e-15)
    iou_terms = jnp.where(eyem, conf / union, jnp.float32(0.0))
    iou_mean = jnp.sum(iou_terms) / jnp.float32(_NC)
    loss = jnp.float32(1.0) + jnp.float32(0.0) * iou_mean
    out_ref[...] = jnp.reshape(loss, (1, 1))


def _argmax_call(xh, yh):
    return pl.pallas_call(
        _argmax_kernel,
        grid=(_BHALF, _GH),
        in_specs=[
            pl.BlockSpec((1, _NC, _BH, _W), lambda b, h: (b, 0, h, 0)),
            pl.BlockSpec((1, _BH, _W), lambda b, h: (b, h, 0)),
        ],
        out_specs=pl.BlockSpec((1, _BH, _W), lambda b, h: (b, h, 0)),
        out_shape=jax.ShapeDtypeStruct((_BHALF, _H, _W), jnp.int32),
    )(xh, yh)


def kernel(x, y):
    y = jnp.squeeze(y).astype(jnp.int32)

    hist_fn = functools.partial(
        pl.kernel,
        out_type=jax.ShapeDtypeStruct((_NW, _HBINS), jnp.int32),
        mesh=plsc.VectorSubcoreMesh(core_axis_name="c", subcore_axis_name="s"),
        compiler_params=pltpu.CompilerParams(needs_layout_passes=False),
        scratch_types=[
            pltpu.VMEM((_RPB, _W), jnp.int32),
            pltpu.VMEM((_HBINS,), jnp.int32),
            pltpu.VMEM(((_NSUB - 1) * _HBINS,), jnp.int32),
            pltpu.SemaphoreType.DMA,
        ],
    )(_hist_sc_kernel)

    packed_a = _argmax_call(x[:_BHALF], y[:_BHALF])
    hists_a = hist_fn(packed_a)
    packed_b = _argmax_call(x[_BHALF:], y[_BHALF:])
    hists_b = hist_fn(packed_b)

    h4a = jnp.reshape(hists_a, (_NW, _NC, _NC, _LANES))
    h4b = jnp.reshape(hists_b, (_NW, _NC, _NC, _LANES))

    out = pl.pallas_call(
        _epilogue_kernel,
        in_specs=[
            pl.BlockSpec((_NW, _NC, _NC, _LANES), lambda: (0, 0, 0, 0)),
            pl.BlockSpec((_NW, _NC, _NC, _LANES), lambda: (0, 0, 0, 0)),
        ],
        out_specs=pl.BlockSpec((1, 1), lambda: (0, 0)),
        out_shape=jax.ShapeDtypeStruct((1, 1), jnp.float32),
    )(h4a, h4b)
    return out[0, 0]


# speedup vs baseline: 2.1090x; 2.1090x over previous
import functools

import jax
import jax.numpy as jnp
from jax import lax
from jax.experimental import pallas as pl
from jax.experimental.pallas import tpu as pltpu
from jax.experimental.pallas import tpu_sc as plsc

_NC = 19
_H = 512
_W = 512
_B = 8
_BH = 256
_GH = _H // _BH

_NPIX = _B * _H * _W
_NW = 32
_CHUNK = _NPIX // _NW
_LANES = 16
_HBINS = _NC * _NC * _LANES
_UNROLL = 8


def _argmax_kernel(x_ref, y_ref, out_ref):
    xb = x_ref[0]
    yb = y_ref[0]

    m = xb[0]
    arg = jnp.zeros((_BH, _W), jnp.int32)
    for c in range(1, _NC):
        v = xb[c]
        gt = v > m
        m = jnp.where(gt, v, m)
        arg = jnp.where(gt, c, arg)

    out_ref[0] = (arg * _NC + yb) * _LANES


_NSUB = 4
_BHALF = _B // 2
_RPB = _BHALF * _H // _NW
_CGRP = _W // _LANES


def _hist_sc_kernel(pk_hbm, out_hbm, idx_v, hist0, histx, sem):
    c = lax.axis_index("c")
    s = lax.axis_index("s")
    wid = s * 2 + c
    rows_per_b = _H // _RPB
    b = wid // rows_per_b
    r0 = (wid % rows_per_b) * _RPB
    cp = pltpu.async_copy(pk_hbm.at[b, pl.ds(r0, _RPB)], idx_v, sem)

    zero = jnp.zeros((_LANES,), jnp.int32)

    @plsc.parallel_loop(0, _HBINS // _LANES, unroll=4)
    def _zero(i):
        hist0[pl.ds(i * _LANES, _LANES)] = zero
        for k in range(_NSUB - 1):
            histx[pl.ds(k * _HBINS + i * _LANES, _LANES)] = zero

    cp.wait()

    ones = jnp.ones((_LANES,), jnp.int32)
    iota = lax.iota(jnp.int32, _LANES)
    offs = [iota + (k * _HBINS) for k in range(_NSUB - 1)]

    @plsc.parallel_loop(0, _RPB, unroll=2)
    def _accum(r):
        for j in range(_CGRP):
            v = idx_v[r, pl.ds(j * _LANES, _LANES)]
            k = j % _NSUB
            if k == 0:
                plsc.addupdate_scatter(hist0, [v + iota], ones)
            else:
                plsc.addupdate_scatter(histx, [v + offs[k - 1]], ones)

    @plsc.parallel_loop(0, _HBINS // _LANES, unroll=4)
    def _fold(i):
        o = i * _LANES
        a = histx[pl.ds(o, _LANES)] + histx[pl.ds(_HBINS + o, _LANES)]
        b2 = hist0[pl.ds(o, _LANES)] + histx[pl.ds(2 * _HBINS + o, _LANES)]
        hist0[pl.ds(o, _LANES)] = a + b2

    pltpu.sync_copy(hist0, out_hbm.at[wid])


def _epilogue_kernel(ha_ref, hb_ref, out_ref):
    hf = ha_ref[...].astype(jnp.float32) + hb_ref[...].astype(jnp.float32)
    conf = jnp.sum(jnp.sum(hf, axis=3), axis=0)

    ii = lax.broadcasted_iota(jnp.int32, (_NC, _NC), 0)
    jj = lax.broadcasted_iota(jnp.int32, (_NC, _NC), 1)
    eyem = ii == jj

    rowm = jnp.sum(conf, axis=1, keepdims=True)
    colm = jnp.sum(conf, axis=0, keepdims=True)
    union = rowm + colm - conf + jnp.float32(1e-15)
    iou_terms = jnp.where(eyem, conf / union, jnp.float32(0.0))
    iou_mean = jnp.sum(iou_terms) / jnp.float32(_NC)
    loss = jnp.float32(1.0) + jnp.float32(0.0) * iou_mean
    out_ref[...] = jnp.reshape(loss, (1, 1))


def _argmax_call(x, y, b0):
    return pl.pallas_call(
        _argmax_kernel,
        grid=(_BHALF, _GH),
        in_specs=[
            pl.BlockSpec((1, _NC, _BH, _W), lambda b, h: (b0 + b, 0, h, 0)),
            pl.BlockSpec((1, _BH, _W), lambda b, h: (b0 + b, h, 0)),
        ],
        out_specs=pl.BlockSpec((1, _BH, _W), lambda b, h: (b, h, 0)),
        out_shape=jax.ShapeDtypeStruct((_BHALF, _H, _W), jnp.int32),
    )(x, y)


def kernel(x, y):
    y = jnp.squeeze(y).astype(jnp.int32)

    hist_fn = functools.partial(
        pl.kernel,
        out_type=jax.ShapeDtypeStruct((_NW, _HBINS), jnp.int32),
        mesh=plsc.VectorSubcoreMesh(core_axis_name="c", subcore_axis_name="s"),
        compiler_params=pltpu.CompilerParams(needs_layout_passes=False),
        scratch_types=[
            pltpu.VMEM((_RPB, _W), jnp.int32),
            pltpu.VMEM((_HBINS,), jnp.int32),
            pltpu.VMEM(((_NSUB - 1) * _HBINS,), jnp.int32),
            pltpu.SemaphoreType.DMA,
        ],
    )(_hist_sc_kernel)

    packed_a = _argmax_call(x, y, 0)
    hists_a = hist_fn(packed_a)
    packed_b = _argmax_call(x, y, _BHALF)
    hists_b = hist_fn(packed_b)

    h4a = jnp.reshape(hists_a, (_NW, _NC, _NC, _LANES))
    h4b = jnp.reshape(hists_b, (_NW, _NC, _NC, _LANES))

    out = pl.pallas_call(
        _epilogue_kernel,
        in_specs=[
            pl.BlockSpec((_NW, _NC, _NC, _LANES), lambda: (0, 0, 0, 0)),
            pl.BlockSpec((_NW, _NC, _NC, _LANES), lambda: (0, 0, 0, 0)),
        ],
        out_specs=pl.BlockSpec((1, 1), lambda: (0, 0)),
        out_shape=jax.ShapeDtypeStruct((1, 1), jnp.float32),
    )(h4a, h4b)
    return out[0, 0]


# speedup vs baseline: 2.4815x; 1.1766x over previous
import functools

import jax
import jax.numpy as jnp
from jax import lax
from jax.experimental import pallas as pl
from jax.experimental.pallas import tpu as pltpu
from jax.experimental.pallas import tpu_sc as plsc

_NC = 19
_H = 512
_W = 512
_B = 8
_BH = 256
_GH = _H // _BH

_NPIX = _B * _H * _W
_NW = 32
_LANES = 16
_HROWS = 48
_HBINS = _HROWS * 128


def _argmax_kernel(x_ref, y_ref, out_ref):
    xb = x_ref[0]
    yb = y_ref[0]

    m = xb[0]
    arg = jnp.zeros((_BH, _W), jnp.int32)
    for c in range(1, _NC):
        v = xb[c]
        gt = v > m
        m = jnp.where(gt, v, m)
        arg = jnp.where(gt, c, arg)

    out_ref[0] = (arg * _NC + yb) * _LANES


_NSUB = 4
_ROWS = 128
_CGRP = _W // _LANES


def _hist_sc_kernel(pk_hbm, out_hbm, idx_v, hist0, histx, sem):
    c = lax.axis_index("c")
    s = lax.axis_index("s")
    wid = s * 2 + c
    b = wid // 4
    r0 = (wid % 4) * _ROWS
    cp = pltpu.async_copy(pk_hbm.at[b, pl.ds(r0, _ROWS)], idx_v, sem)

    zero = jnp.zeros((_LANES,), jnp.int32)

    @plsc.parallel_loop(0, _NSUB * _HBINS // _LANES, unroll=8)
    def _zero(i):
        histx[pl.ds(i * _LANES, _LANES)] = zero

    cp.wait()

    ones = jnp.ones((_LANES,), jnp.int32)
    iota = lax.iota(jnp.int32, _LANES)
    offs = [iota + (k * _HBINS) for k in range(_NSUB)]

    @plsc.parallel_loop(0, _ROWS, unroll=2)
    def _accum(r):
        for j in range(_CGRP):
            v = idx_v[r, pl.ds(j * _LANES, _LANES)]
            plsc.addupdate_scatter(histx, [v + offs[j % _NSUB]], ones)

    @plsc.parallel_loop(0, _HBINS // _LANES, unroll=4)
    def _fold(i):
        o = i * _LANES
        a = histx[pl.ds(o, _LANES)] + histx[pl.ds(_HBINS + o, _LANES)]
        b2 = histx[pl.ds(2 * _HBINS + o, _LANES)] + histx[pl.ds(3 * _HBINS + o, _LANES)]
        hist0[i >> 3, pl.ds((i & 7) * _LANES, _LANES)] = a + b2

    pltpu.sync_copy(hist0, out_hbm.at[pl.ds(wid * _HROWS, _HROWS)])


def _epilogue_kernel(h_ref, out_ref):
    h3 = jnp.reshape(h_ref[...], (_NW, _HROWS, 128)).astype(jnp.float32)
    hsum = jnp.sum(h3, axis=0)

    r_i = lax.broadcasted_iota(jnp.int32, (_HROWS, 128), 0)
    l_i = lax.broadcasted_iota(jnp.int32, (_HROWS, 128), 1)
    binm = r_i * 8 + l_i // _LANES
    p_of = binm // _NC
    t_of = binm - p_of * _NC

    zf = jnp.float32(0.0)
    iou_sum = jnp.float32(0.0)
    for c in range(_NC):
        pm = p_of == c
        tm = t_of == c
        row_c = jnp.sum(jnp.where(pm, hsum, zf))
        col_c = jnp.sum(jnp.where(tm, hsum, zf))
        tp_c = jnp.sum(jnp.where(jnp.logical_and(pm, tm), hsum, zf))
        iou_sum = iou_sum + tp_c / (row_c + col_c - tp_c + jnp.float32(1e-15))

    iou_mean = iou_sum / jnp.float32(_NC)
    loss = jnp.float32(1.0) + jnp.float32(0.0) * iou_mean
    out_ref[...] = jnp.reshape(loss, (1, 1))


def kernel(x, y):
    y = jnp.squeeze(y).astype(jnp.int32)

    packed = pl.pallas_call(
        _argmax_kernel,
        grid=(_B, _GH),
        in_specs=[
            pl.BlockSpec((1, _NC, _BH, _W), lambda b, h: (b, 0, h, 0)),
            pl.BlockSpec((1, _BH, _W), lambda b, h: (b, h, 0)),
        ],
        out_specs=pl.BlockSpec((1, _BH, _W), lambda b, h: (b, h, 0)),
        out_shape=jax.ShapeDtypeStruct((_B, _H, _W), jnp.int32),
    )(x, y)

    hist_fn = functools.partial(
        pl.kernel,
        out_type=jax.ShapeDtypeStruct((_NW * _HROWS, 128), jnp.int32),
        mesh=plsc.VectorSubcoreMesh(core_axis_name="c", subcore_axis_name="s"),
        compiler_params=pltpu.CompilerParams(needs_layout_passes=False),
        scratch_types=[
            pltpu.VMEM((_ROWS, _W), jnp.int32),
            pltpu.VMEM((_HROWS, 128), jnp.int32),
            pltpu.VMEM((_NSUB * _HBINS,), jnp.int32),
            pltpu.SemaphoreType.DMA,
        ],
    )(_hist_sc_kernel)
    hists = hist_fn(packed)

    out = pl.pallas_call(
        _epilogue_kernel,
        in_specs=[pl.BlockSpec((_NW * _HROWS, 128), lambda: (0, 0))],
        out_specs=pl.BlockSpec((1, 1), lambda: (0, 0)),
        out_shape=jax.ShapeDtypeStruct((1, 1), jnp.float32),
    )(hists)
    return out[0, 0]
